# baseline (speedup 1.0000x reference)
import functools

import jax
import jax.numpy as jnp
from jax import lax
from jax.experimental import pallas as pl
from jax.experimental.pallas import tpu as pltpu
from jax.experimental.pallas import tpu_sc as plsc

L = 2
N = 262144
D = 128
B = 16384

NC = 2
NS = 16
NW = NC * NS
SEGS_PER_W = B // NW
CHUNK_SEGS = 8
MAX_SEG = 16
CHUNK_ROWS = 144
STAGE_SEGS = SEGS_PER_W // 2
N_CHUNKS = SEGS_PER_W // CHUNK_SEGS
ROOT_CHUNK = 128

LANES = 16
NGRP = D // LANES

TAIL_BR = 512
TAIL_NBLK = N // TAIL_BR
FINAL_BROW = 1024


def _sc_body(table, offs, sizes, idxt, root_out, pool_out,
             offs_v, size_v, idx0_v, idx1_v,
             rows0, rows1, rows0b, rows1b, stage, sem0, sem1, sem2, sem3):
    wid = lax.axis_index("s") * NC + lax.axis_index("c")
    seg_base = pl.multiple_of(wid * SEGS_PER_W, SEGS_PER_W)

    pltpu.sync_copy(offs.at[pl.ds(seg_base, SEGS_PER_W + LANES)], offs_v)
    pltpu.sync_copy(sizes.at[pl.ds(seg_base, SEGS_PER_W + LANES)], size_v)

    for rc in range(SEGS_PER_W // ROOT_CHUNK):
        base = pl.multiple_of(seg_base + rc * ROOT_CHUNK, ROOT_CHUNK)
        pltpu.sync_copy(idxt.at[pl.ds(base, ROOT_CHUNK)], idx0_v)
        for g in range(ROOT_CHUNK // LANES):
            s = pl.ds(g * LANES, LANES)
            idx1_v[s] = idx0_v[s] + N
        rr0 = rows0.at[pl.ds(0, ROOT_CHUNK)]
        rr1 = rows1.at[pl.ds(0, ROOT_CHUNK)]
        cp0 = pltpu.make_async_copy(table.at[idx0_v], rr0, sem0)
        cp1 = pltpu.make_async_copy(table.at[idx1_v], rr1, sem1)
        cp0.start()
        cp1.start()
        cp0.wait()
        cp1.wait()

        def _radd(r, carry):
            for g in range(NGRP):
                s = pl.ds(g * LANES, LANES)
                rows0[r, s] = rows0[r, s] + rows1[r, s]
            return carry

        lax.fori_loop(0, ROOT_CHUNK, _radd, 0)
        pltpu.sync_copy(rr0, root_out.at[pl.ds(base, ROOT_CHUNK)])

    def _chunk_r(c):
        start_raw = offs_v[pl.ds(c * CHUNK_SEGS, LANES)][0]
        r = jnp.minimum((start_raw // 8) * 8, N - CHUNK_ROWS)
        return pl.multiple_of(r, 8)

    def _start(c, bufs, sems):
        r = _chunk_r(c)
        pltpu.make_async_copy(
            table.at[pl.ds(r, CHUNK_ROWS)], bufs[0], sems[0]).start()
        pltpu.make_async_copy(
            table.at[pl.ds(N + r, CHUNK_ROWS)], bufs[1], sems[1]).start()

    def _finish(c, bufs, sems, half):
        r = _chunk_r(c)
        s0 = c * CHUNK_SEGS
        pltpu.make_async_copy(
            table.at[pl.ds(r, CHUNK_ROWS)], bufs[0], sems[0]).wait()
        pltpu.make_async_copy(
            table.at[pl.ds(N + r, CHUNK_ROWS)], bufs[1], sems[1]).wait()

        def _seg(k, carry2):
            ls2 = s0 + k
            off_raw = offs_v[pl.ds(ls2, LANES)][0]
            size_k = size_v[pl.ds(ls2, LANES)][0]
            off_k = off_raw - r
            seg_id = seg_base + ls2
            count = jnp.where(seg_id == B - 1, N - off_raw, size_k)
            countf = count.astype(jnp.float32)
            numv = jnp.full((LANES,), jnp.where(count > 0, 1.0, 0.0),
                            jnp.float32)
            recip = numv / jnp.maximum(jnp.full((LANES,), countf), 1.0)

            acc = [jnp.zeros((LANES,), jnp.float32) for _ in range(NGRP)]
            for j in range(MAX_SEG):
                mj = jnp.where(j < size_k, 1.0, 0.0)
                mv = jnp.full((LANES,), mj, jnp.float32)
                rr = jnp.minimum(off_k + j, CHUNK_ROWS - 1)
                for g in range(NGRP):
                    s = pl.ds(g * LANES, LANES)
                    acc[g] = acc[g] + mv * (bufs[0][rr, s] + bufs[1][rr, s])
            for g in range(NGRP):
                stage[ls2 - half * STAGE_SEGS,
                      pl.ds(g * LANES, LANES)] = acc[g] * recip
            return carry2

        lax.fori_loop(0, CHUNK_SEGS, _seg, 0)

    bufs_a = (rows0, rows1)
    bufs_b = (rows0b, rows1b)
    sems_a = (sem0, sem1)
    sems_b = (sem2, sem3)
    half_chunks = N_CHUNKS // 2
    _start(0, bufs_a, sems_a)
    for half in range(2):
        base_c = half * half_chunks

        def _pair(i, carry0, base_c=base_c, half=half):
            c0 = base_c + 2 * i
            _start(c0 + 1, bufs_b, sems_b)
            _finish(c0, bufs_a, sems_a, half)

            @pl.when(c0 + 2 < N_CHUNKS)
            def _():
                _start(c0 + 2, bufs_a, sems_a)

            _finish(c0 + 1, bufs_b, sems_b, half)
            return carry0

        lax.fori_loop(0, half_chunks // 2, _pair, 0)
        pltpu.sync_copy(
            stage,
            pool_out.at[pl.ds(
                pl.multiple_of(seg_base + half * STAGE_SEGS, STAGE_SEGS),
                STAGE_SEGS)])


@functools.cache
def _sc_pool_fn():
    return functools.partial(
        pl.kernel,
        out_type=[
            jax.ShapeDtypeStruct((B, D), jnp.float32),
            jax.ShapeDtypeStruct((B, D), jnp.float32),
        ],
        mesh=plsc.VectorSubcoreMesh(
            core_axis_name="c", subcore_axis_name="s",
            num_cores=NC, num_subcores=NS),
        scratch_types=[
            pltpu.VMEM((SEGS_PER_W + LANES,), jnp.int32),
            pltpu.VMEM((SEGS_PER_W + LANES,), jnp.int32),
            pltpu.VMEM((ROOT_CHUNK,), jnp.int32),
            pltpu.VMEM((ROOT_CHUNK,), jnp.int32),
            pltpu.VMEM((CHUNK_ROWS, D), jnp.float32),
            pltpu.VMEM((CHUNK_ROWS, D), jnp.float32),
            pltpu.VMEM((CHUNK_ROWS, D), jnp.float32),
            pltpu.VMEM((CHUNK_ROWS, D), jnp.float32),
            pltpu.VMEM((STAGE_SEGS, D), jnp.float32),
            pltpu.SemaphoreType.DMA,
            pltpu.SemaphoreType.DMA,
            pltpu.SemaphoreType.DMA,
            pltpu.SemaphoreType.DMA,
        ],
    )(_sc_body)


def _tail_body(scal_ref, table_ref, out_ref):
    i = pl.program_id(0)
    blk0 = scal_ref[0]
    j = jnp.where(i < TAIL_NBLK,
                  jnp.maximum(i, blk0),
                  jnp.maximum(i, TAIL_NBLK + blk0))

    @pl.when(i == 0)
    def _():
        out_ref[...] = jnp.zeros_like(out_ref)

    @pl.when(i == j)
    def _():
        total = scal_ref[1]
        r = j * TAIL_BR + lax.broadcasted_iota(jnp.int32, (TAIL_BR, 1), 0)
        valid = ((r >= total) & (r < N)) | (r >= N + total)
        x = jnp.where(valid, table_ref[...], 0.0)
        out_ref[...] += x.reshape(TAIL_BR // 8, 8, D).sum(axis=0)


def _tail_index_map(i, scal_ref):
    blk0 = scal_ref[0]
    return (jnp.where(i < TAIL_NBLK,
                      jnp.maximum(i, blk0),
                      jnp.maximum(i, TAIL_NBLK + blk0)), 0)


def _tail_call(scal, table):
    return pl.pallas_call(
        _tail_body,
        grid_spec=pltpu.PrefetchScalarGridSpec(
            num_scalar_prefetch=1,
            grid=(2 * TAIL_NBLK,),
            in_specs=[pl.BlockSpec((TAIL_BR, D), _tail_index_map)],
            out_specs=pl.BlockSpec((8, D), lambda i, s: (0, 0)),
        ),
        out_shape=jax.ShapeDtypeStruct((8, D), jnp.float32),
    )(scal, table)


def _final_body(inv_cl_ref, root_ref, pool_ref, tail_ref, a1_ref, a2_ref,
                b_ref, sc_ref, of_ref, out_ref):
    i = pl.program_id(0)
    pool = pool_ref[...]
    tail = tail_ref[...].sum(axis=0, keepdims=True)
    gr = i * FINAL_BROW + lax.broadcasted_iota(jnp.int32, (FINAL_BROW, 1), 0)
    pool = pool + jnp.where(gr == B - 1, tail * inv_cl_ref[0], 0.0)
    h = (jnp.dot(root_ref[...], a1_ref[...],
                 preferred_element_type=jnp.float32)
         + jnp.dot(pool, a2_ref[...], preferred_element_type=jnp.float32)
         + b_ref[...])
    h = jnp.maximum(h, 0.0)
    mean = jnp.mean(h, axis=1, keepdims=True)
    hc = h - mean
    var = jnp.mean(hc * hc, axis=1, keepdims=True) + 1e-9
    out_ref[...] = hc * sc_ref[...] * lax.rsqrt(var) + of_ref[...]


def _final_call(inv_cl, root, pool, tail, a1, a2, bb, sc, of):
    nblk = B // FINAL_BROW
    return pl.pallas_call(
        _final_body,
        grid_spec=pltpu.PrefetchScalarGridSpec(
            num_scalar_prefetch=1,
            grid=(nblk,),
            in_specs=[
                pl.BlockSpec((FINAL_BROW, D), lambda i, s: (i, 0)),
                pl.BlockSpec((FINAL_BROW, D), lambda i, s: (i, 0)),
                pl.BlockSpec((8, D), lambda i, s: (0, 0)),
                pl.BlockSpec((D, D), lambda i, s: (0, 0)),
                pl.BlockSpec((D, D), lambda i, s: (0, 0)),
                pl.BlockSpec((1, D), lambda i, s: (0, 0)),
                pl.BlockSpec((1, D), lambda i, s: (0, 0)),
                pl.BlockSpec((1, D), lambda i, s: (0, 0)),
            ],
            out_specs=pl.BlockSpec((FINAL_BROW, D), lambda i, s: (i, 0)),
        ),
        out_shape=jax.ShapeDtypeStruct((B, D), jnp.float32),
    )(inv_cl, root, pool, tail, a1, a2, bb, sc, of)


def kernel(feats_in_l, idx_targets, sizes_subg, W, b, scale, offset):
    table = feats_in_l.reshape(L * N, D)
    cum = jnp.cumsum(sizes_subg).astype(jnp.int32)
    total = cum[-1]
    offs = jnp.concatenate(
        [jnp.zeros((1,), jnp.int32), cum[:-1]])
    pad = jnp.zeros((LANES,), jnp.int32)
    offs_p = jnp.concatenate([offs, pad])
    sizes_p = jnp.concatenate([sizes_subg, pad])

    root, pool = _sc_pool_fn()(table, offs_p, sizes_p, idx_targets)

    blk0 = jnp.minimum(total // TAIL_BR, TAIL_NBLK - 1)
    tail8 = _tail_call(jnp.stack([blk0, total]).astype(jnp.int32), table)

    count_last = (N - offs[-1]).astype(jnp.float32)
    inv_cl = jnp.where(count_last > 0, 1.0 / count_last, 0.0)

    a1 = jnp.transpose(W[:, :D])
    a2 = jnp.transpose(W[:, D:])
    return _final_call(inv_cl[None].astype(jnp.float32), root, pool, tail8,
                       a1, a2, b[None], scale[None], offset[None])


# speedup vs baseline: 1.7613x; 1.7613x over previous
import functools

import jax
import jax.numpy as jnp
from jax import lax
from jax.experimental import pallas as pl
from jax.experimental.pallas import tpu as pltpu
from jax.experimental.pallas import tpu_sc as plsc

L = 2
N = 262144
D = 128
B = 16384

NC = 2
NS = 16
NW = NC * NS
SEGS_PER_W = B // NW
CHUNK_SEGS = 8
MAX_SEG = 16
CHUNK_ROWS = 144
STAGE_SEGS = SEGS_PER_W // 2
N_CHUNKS = SEGS_PER_W // CHUNK_SEGS
ROOT_CHUNK = 128

LANES = 16
NGRP = D // LANES

TAIL_BR = 2048
TAIL_NBLK = N // TAIL_BR
FINAL_BROW = 1024


def _sc_body(table, offs, sizes, idxt, root_out, pool_out,
             offs_v, size_v, idx0_v, idx1_v,
             rows0, rows1, rows0b, rows1b, stage, sem0, sem1, sem2, sem3):
    wid = lax.axis_index("s") * NC + lax.axis_index("c")
    seg_base = pl.multiple_of(wid * SEGS_PER_W, SEGS_PER_W)

    pltpu.sync_copy(offs.at[pl.ds(seg_base, SEGS_PER_W + LANES)], offs_v)
    pltpu.sync_copy(sizes.at[pl.ds(seg_base, SEGS_PER_W + LANES)], size_v)

    for rc in range(SEGS_PER_W // ROOT_CHUNK):
        base = pl.multiple_of(seg_base + rc * ROOT_CHUNK, ROOT_CHUNK)
        pltpu.sync_copy(idxt.at[pl.ds(base, ROOT_CHUNK)], idx0_v)
        for g in range(ROOT_CHUNK // LANES):
            s = pl.ds(g * LANES, LANES)
            idx1_v[s] = idx0_v[s] + N
        rr0 = rows0.at[pl.ds(0, ROOT_CHUNK)]
        rr1 = rows1.at[pl.ds(0, ROOT_CHUNK)]
        cp0 = pltpu.make_async_copy(table.at[idx0_v], rr0, sem0)
        cp1 = pltpu.make_async_copy(table.at[idx1_v], rr1, sem1)
        cp0.start()
        cp1.start()
        cp0.wait()
        cp1.wait()

        def _radd(r, carry):
            for g in range(NGRP):
                s = pl.ds(g * LANES, LANES)
                rows0[r, s] = rows0[r, s] + rows1[r, s]
            return carry

        lax.fori_loop(0, ROOT_CHUNK, _radd, 0)
        pltpu.sync_copy(rr0, root_out.at[pl.ds(base, ROOT_CHUNK)])

    def _chunk_r(c):
        start_raw = offs_v[pl.ds(c * CHUNK_SEGS, LANES)][0]
        r = jnp.minimum((start_raw // 8) * 8, N - CHUNK_ROWS)
        return pl.multiple_of(r, 8)

    def _start(c, bufs, sems):
        r = _chunk_r(c)
        pltpu.make_async_copy(
            table.at[pl.ds(r, CHUNK_ROWS)], bufs[0], sems[0]).start()
        pltpu.make_async_copy(
            table.at[pl.ds(N + r, CHUNK_ROWS)], bufs[1], sems[1]).start()

    def _finish(c, bufs, sems, half):
        r = _chunk_r(c)
        s0 = c * CHUNK_SEGS
        pltpu.make_async_copy(
            table.at[pl.ds(r, CHUNK_ROWS)], bufs[0], sems[0]).wait()
        pltpu.make_async_copy(
            table.at[pl.ds(N + r, CHUNK_ROWS)], bufs[1], sems[1]).wait()

        def _seg(k, carry2):
            ls2 = s0 + k
            off_raw = offs_v[pl.ds(ls2, LANES)][0]
            size_k = size_v[pl.ds(ls2, LANES)][0]
            off_k = off_raw - r
            seg_id = seg_base + ls2
            count = jnp.where(seg_id == B - 1, N - off_raw, size_k)
            countf = count.astype(jnp.float32)
            numv = jnp.full((LANES,), jnp.where(count > 0, 1.0, 0.0),
                            jnp.float32)
            recip = numv / jnp.maximum(jnp.full((LANES,), countf), 1.0)

            acc = [jnp.zeros((LANES,), jnp.float32) for _ in range(NGRP)]
            for j in range(MAX_SEG):
                mj = jnp.where(j < size_k, 1.0, 0.0)
                mv = jnp.full((LANES,), mj, jnp.float32)
                rr = jnp.minimum(off_k + j, CHUNK_ROWS - 1)
                for g in range(NGRP):
                    s = pl.ds(g * LANES, LANES)
                    acc[g] = acc[g] + mv * (bufs[0][rr, s] + bufs[1][rr, s])
            for g in range(NGRP):
                stage[ls2 - half * STAGE_SEGS,
                      pl.ds(g * LANES, LANES)] = acc[g] * recip
            return carry2

        lax.fori_loop(0, CHUNK_SEGS, _seg, 0)

    bufs_a = (rows0, rows1)
    bufs_b = (rows0b, rows1b)
    sems_a = (sem0, sem1)
    sems_b = (sem2, sem3)
    half_chunks = N_CHUNKS // 2
    _start(0, bufs_a, sems_a)
    for half in range(2):
        base_c = half * half_chunks

        def _pair(i, carry0, base_c=base_c, half=half):
            c0 = base_c + 2 * i
            _start(c0 + 1, bufs_b, sems_b)
            _finish(c0, bufs_a, sems_a, half)

            @pl.when(c0 + 2 < N_CHUNKS)
            def _():
                _start(c0 + 2, bufs_a, sems_a)

            _finish(c0 + 1, bufs_b, sems_b, half)
            return carry0

        lax.fori_loop(0, half_chunks // 2, _pair, 0)
        pltpu.sync_copy(
            stage,
            pool_out.at[pl.ds(
                pl.multiple_of(seg_base + half * STAGE_SEGS, STAGE_SEGS),
                STAGE_SEGS)])


@functools.cache
def _sc_pool_fn():
    return functools.partial(
        pl.kernel,
        out_type=[
            jax.ShapeDtypeStruct((B, D), jnp.float32),
            jax.ShapeDtypeStruct((B, D), jnp.float32),
        ],
        mesh=plsc.VectorSubcoreMesh(
            core_axis_name="c", subcore_axis_name="s",
            num_cores=NC, num_subcores=NS),
        scratch_types=[
            pltpu.VMEM((SEGS_PER_W + LANES,), jnp.int32),
            pltpu.VMEM((SEGS_PER_W + LANES,), jnp.int32),
            pltpu.VMEM((ROOT_CHUNK,), jnp.int32),
            pltpu.VMEM((ROOT_CHUNK,), jnp.int32),
            pltpu.VMEM((CHUNK_ROWS, D), jnp.float32),
            pltpu.VMEM((CHUNK_ROWS, D), jnp.float32),
            pltpu.VMEM((CHUNK_ROWS, D), jnp.float32),
            pltpu.VMEM((CHUNK_ROWS, D), jnp.float32),
            pltpu.VMEM((STAGE_SEGS, D), jnp.float32),
            pltpu.SemaphoreType.DMA,
            pltpu.SemaphoreType.DMA,
            pltpu.SemaphoreType.DMA,
            pltpu.SemaphoreType.DMA,
        ],
    )(_sc_body)


def _tail_body(scal_ref, table_ref, out_ref):
    i = pl.program_id(0)
    blk0 = scal_ref[0]
    j = jnp.where(i < TAIL_NBLK,
                  jnp.maximum(i, blk0),
                  jnp.maximum(i, TAIL_NBLK + blk0))

    @pl.when(i == 0)
    def _():
        out_ref[...] = jnp.zeros_like(out_ref)

    @pl.when(i == j)
    def _():
        total = scal_ref[1]
        r = j * TAIL_BR + lax.broadcasted_iota(jnp.int32, (TAIL_BR, 1), 0)
        valid = ((r >= total) & (r < N)) | (r >= N + total)
        x = jnp.where(valid, table_ref[...], 0.0)
        out_ref[...] += x.reshape(TAIL_BR // 8, 8, D).sum(axis=0)


def _tail_index_map(i, scal_ref):
    blk0 = scal_ref[0]
    return (jnp.where(i < TAIL_NBLK,
                      jnp.maximum(i, blk0),
                      jnp.maximum(i, TAIL_NBLK + blk0)), 0)


def _tail_call(scal, table):
    return pl.pallas_call(
        _tail_body,
        grid_spec=pltpu.PrefetchScalarGridSpec(
            num_scalar_prefetch=1,
            grid=(2 * TAIL_NBLK,),
            in_specs=[pl.BlockSpec((TAIL_BR, D), _tail_index_map)],
            out_specs=pl.BlockSpec((8, D), lambda i, s: (0, 0)),
        ),
        out_shape=jax.ShapeDtypeStruct((8, D), jnp.float32),
    )(scal, table)


def _final_body(inv_cl_ref, root_ref, pool_ref, tail_ref, a1_ref, a2_ref,
                b_ref, sc_ref, of_ref, out_ref):
    i = pl.program_id(0)
    pool = pool_ref[...]
    tail = tail_ref[...].sum(axis=0, keepdims=True)
    gr = i * FINAL_BROW + lax.broadcasted_iota(jnp.int32, (FINAL_BROW, 1), 0)
    pool = pool + jnp.where(gr == B - 1, tail * inv_cl_ref[0], 0.0)
    h = (jnp.dot(root_ref[...], a1_ref[...],
                 preferred_element_type=jnp.float32)
         + jnp.dot(pool, a2_ref[...], preferred_element_type=jnp.float32)
         + b_ref[...])
    h = jnp.maximum(h, 0.0)
    mean = jnp.mean(h, axis=1, keepdims=True)
    hc = h - mean
    var = jnp.mean(hc * hc, axis=1, keepdims=True) + 1e-9
    out_ref[...] = hc * sc_ref[...] * lax.rsqrt(var) + of_ref[...]


def _final_call(inv_cl, root, pool, tail, a1, a2, bb, sc, of):
    nblk = B // FINAL_BROW
    return pl.pallas_call(
        _final_body,
        grid_spec=pltpu.PrefetchScalarGridSpec(
            num_scalar_prefetch=1,
            grid=(nblk,),
            in_specs=[
                pl.BlockSpec((FINAL_BROW, D), lambda i, s: (i, 0)),
                pl.BlockSpec((FINAL_BROW, D), lambda i, s: (i, 0)),
                pl.BlockSpec((8, D), lambda i, s: (0, 0)),
                pl.BlockSpec((D, D), lambda i, s: (0, 0)),
                pl.BlockSpec((D, D), lambda i, s: (0, 0)),
                pl.BlockSpec((1, D), lambda i, s: (0, 0)),
                pl.BlockSpec((1, D), lambda i, s: (0, 0)),
                pl.BlockSpec((1, D), lambda i, s: (0, 0)),
            ],
            out_specs=pl.BlockSpec((FINAL_BROW, D), lambda i, s: (i, 0)),
        ),
        out_shape=jax.ShapeDtypeStruct((B, D), jnp.float32),
    )(inv_cl, root, pool, tail, a1, a2, bb, sc, of)


def kernel(feats_in_l, idx_targets, sizes_subg, W, b, scale, offset):
    table = feats_in_l.reshape(L * N, D)
    cum = jnp.cumsum(sizes_subg).astype(jnp.int32)
    total = cum[-1]
    offs = jnp.concatenate(
        [jnp.zeros((1,), jnp.int32), cum[:-1]])
    pad = jnp.zeros((LANES,), jnp.int32)
    offs_p = jnp.concatenate([offs, pad])
    sizes_p = jnp.concatenate([sizes_subg, pad])

    root, pool = _sc_pool_fn()(table, offs_p, sizes_p, idx_targets)

    blk0 = jnp.minimum(total // TAIL_BR, TAIL_NBLK - 1)
    tail8 = _tail_call(jnp.stack([blk0, total]).astype(jnp.int32), table)

    count_last = (N - offs[-1]).astype(jnp.float32)
    inv_cl = jnp.where(count_last > 0, 1.0 / count_last, 0.0)

    a1 = jnp.transpose(W[:, :D])
    a2 = jnp.transpose(W[:, D:])
    return _final_call(inv_cl[None].astype(jnp.float32), root, pool, tail8,
                       a1, a2, b[None], scale[None], offset[None])
